# baseline (speedup 1.0000x reference)
import functools

import jax
import jax.numpy as jnp
from jax import lax
from jax.experimental import pallas as pl
from jax.experimental.pallas import tpu as pltpu
from jax.experimental.pallas import tpu_sc as plsc

_NUM_MODELS = 1000000
_DIM = 64
_TEXT_DIM = 128
_BATCH = 16384

_INFO = plsc.get_sparse_core_info()
_NC, _NS = _INFO.num_cores, _INFO.num_subcores
_NW = _NC * _NS
_NTC = (_NUM_MODELS + 127) // 128
_TC_PER_W = (_NTC + _NW - 1) // _NW
_WCOLS = 4
_NWIN = (_TC_PER_W + _WCOLS - 1) // _WCOLS
_WLCAP = 640
_WLG = _WLCAP // 16
_MAXC0 = (_NTC - 1 - _WCOLS) * 128
_EDGE0 = (_NTC - 1) * 128
_EDGEN = _NUM_MODELS - _EDGE0
_BCH = 1024
_TCHUNK = 16

_EDGE_MLOC = 255

_I16 = lambda: lax.iota(jnp.int32, 16)


def _sc_sweep_kernel(xt_hbm, idx_hbm, t2_hbm, edge_hbm, out_hbm,
                     wlp, wlp2, win_j, rtab, rows_t, twin, twin_b, b_tbl,
                     buf_a, buf_b, buf_e, ibuf_a,
                     sem_a, sem_b, sem_i, sem_t, sem_u):
    cidx = lax.axis_index("c")
    sid = lax.axis_index("s")
    wid = sid * _NC + cidx
    lo = wid * _TC_PER_W
    hi = jnp.minimum(lo + _TC_PER_W, _NTC)

    def fire(w, buf, sem):
        c0 = pl.multiple_of(jnp.minimum((lo + w * _WCOLS) * 128, _MAXC0), 128)
        return pltpu.async_copy(
            xt_hbm.at[:, pl.ds(c0, _WCOLS * 128)], buf, sem)

    fire(0, buf_a, sem_a)
    fire(1, buf_b, sem_b)

    def build_chunk(ib, c, cnt):
        def grp(g, cnt):
            iv = ib[pl.ds(g * 16, 16)]
            mv = iv >> 7
            mask = (mv >= lo) & (mv < hi)
            mloc = jnp.where(mv == _NTC - 1, _EDGE_MLOC, mv - lo)
            p = (mloc << 21) | ((c * _BCH + g * 16 + _I16()) << 7) | (iv & 127)
            pos = jnp.minimum(
                cnt + plsc.cumsum(mask.astype(jnp.int32)) - 1, _WLCAP - 1)
            plsc.store_scatter(wlp, [pos >> 7, pos & 127], p, mask=mask)
            return cnt + plsc.all_reduce_population_count(mask)

        return lax.fori_loop(0, _BCH // 16, grp, cnt)

    cnt = jnp.zeros(16, jnp.int32)
    for c in range(_BATCH // _BCH):
        pltpu.sync_copy(idx_hbm.at[pl.ds(c * _BCH, _BCH)], ibuf_a)
        cnt = build_chunk(ibuf_a, c, cnt)

    def rpass(src, dst, bit):
        def czero(g, z):
            p = src[g >> 3, pl.ds((g & 7) * 16, 16)]
            valid = (g * 16 + _I16()) < cnt
            m0 = (((p >> bit) & 1) == 0) & valid
            return z + plsc.all_reduce_population_count(m0)

        z = lax.fori_loop(0, _WLG, czero, jnp.zeros(16, jnp.int32))

        def scat(g, cs):
            c0, c1 = cs
            p = src[g >> 3, pl.ds((g & 7) * 16, 16)]
            valid = (g * 16 + _I16()) < cnt
            b1 = ((p >> bit) & 1) == 1
            m0 = valid & (~b1)
            m1 = valid & b1
            pos0 = jnp.minimum(c0 + plsc.cumsum(m0.astype(jnp.int32)) - 1,
                               _WLCAP - 1)
            pos1 = jnp.minimum(z + c1 + plsc.cumsum(m1.astype(jnp.int32)) - 1,
                               _WLCAP - 1)
            plsc.store_scatter(dst, [pos0 >> 7, pos0 & 127], p, mask=m0)
            plsc.store_scatter(dst, [pos1 >> 7, pos1 & 127], p, mask=m1)
            return (c0 + plsc.all_reduce_population_count(m0),
                    c1 + plsc.all_reduce_population_count(m1))

        lax.fori_loop(0, _WLG, scat,
                      (jnp.zeros(16, jnp.int32), jnp.zeros(16, jnp.int32)))

    rpass(wlp, wlp2, 26)
    rpass(wlp2, wlp, 27)
    rpass(wlp, wlp2, 28)
    wl = wlp2

    def rcount(g, rc):
        p = wl[g >> 3, pl.ds((g & 7) * 16, 16)]
        valid = (g * 16 + _I16()) < cnt
        rv = (p >> 26) & 7
        return tuple(
            rc[r] + plsc.all_reduce_population_count(valid & (rv == r))
            for r in range(8))

    rc = lax.fori_loop(0, _WLG, rcount,
                       tuple(jnp.zeros(16, jnp.int32) for _ in range(8)))
    rstart = [jnp.zeros(16, jnp.int32)]
    for r in range(8):
        rstart.append(rstart[r] + rc[r])
    rvec = jnp.zeros(16, jnp.int32)
    for r in range(9):
        rvec = jnp.where(_I16() == r, rstart[r], rvec)
    rtab[...] = rvec

    def scan_hits(lo_m, hi_m, g_lo, g_hi):
        def body(g, wcnt):
            wm = wl[g >> 3, pl.ds((g & 7) * 16, 16)] >> 21
            valid = (g * 16 + _I16()) < cnt
            mask = (wm >= lo_m) & (wm < hi_m) & valid
            pos = jnp.minimum(
                wcnt + plsc.cumsum(mask.astype(jnp.int32)) - 1, 31)
            plsc.store_scatter(win_j, [pos], g * 16 + _I16(), mask=mask)
            return wcnt + plsc.all_reduce_population_count(mask)

        return lax.fori_loop(g_lo, g_hi, body, jnp.zeros(16, jnp.int32))

    def extract(buf, wcnt, wbase, edge):
        for g in range(2):
            jv = win_j[pl.ds(g * 16, 16)]
            active = (g * 16 + _I16()) < wcnt
            jv = jnp.where(active, jv, 0)
            p = plsc.load_gather(wl, [jv >> 7, jv & 127])
            if edge:
                colloc = p & 127
            else:
                colloc = (((p >> 21) - wbase) * 128) + (p & 127)
            colloc = jnp.where(active, colloc, 0)

            def dstep(i, _):
                d0 = jnp.full((16,), 2 * i, jnp.int32)
                d1 = d0 + 1
                v0 = plsc.load_gather(buf, [d0, colloc])
                v1 = plsc.load_gather(buf, [d1, colloc])
                plsc.store_scatter(rows_t, [d0, jv], v0, mask=active)
                plsc.store_scatter(rows_t, [d1, jv], v1, mask=active)
                return 0

            lax.fori_loop(0, _DIM // 2, dstep, 0)

    def process(w, buf):
        wlo = lo + w * _WCOLS
        c0 = pl.multiple_of(jnp.minimum(wlo * 128, _MAXC0), 128)
        rsp = jnp.full((16,), w >> 3, jnp.int32)
        g_lo = (plsc.load_gather(rtab, [rsp]) >> 4)[0]
        g_hi = ((plsc.load_gather(rtab, [rsp + 1]) + 15) >> 4)[0]
        wcnt = scan_hits(wlo - lo, jnp.minimum(wlo + _WCOLS, _NTC - 1) - lo,
                         g_lo, g_hi)
        extract(buf, wcnt, c0 // 128 - lo, edge=False)

    def body(i, _):
        w0 = i * 2
        pltpu.make_async_copy(
            xt_hbm.at[:, pl.ds(0, _WCOLS * 128)], buf_a, sem_a).wait()
        process(w0, buf_a)

        @pl.when(w0 + 2 < _NWIN)
        def _():
            fire(w0 + 2, buf_a, sem_a)

        pltpu.make_async_copy(
            xt_hbm.at[:, pl.ds(0, _WCOLS * 128)], buf_b, sem_b).wait()
        process(w0 + 1, buf_b)

        @pl.when(w0 + 3 < _NWIN)
        def _():
            fire(w0 + 3, buf_b, sem_b)

        return 0

    lax.fori_loop(0, _NWIN // 2, body, 0)

    pltpu.sync_copy(edge_hbm, buf_e)
    ecnt = scan_hits(_EDGE_MLOC, _EDGE_MLOC + 1,
                     (rstart[7] >> 4)[0], ((rstart[8] + 15) >> 4)[0])
    extract(buf_e, ecnt, 0, edge=True)

    def unpack_b(g, _):
        j0 = g * 16
        valid = (j0 + _I16()) < cnt
        p = wl[g >> 3, pl.ds((g & 7) * 16, 16)]
        bv = jnp.where(valid, (p >> 7) & 16383, 0)
        b_tbl[g, :] = bv
        return 0

    lax.fori_loop(0, _WLCAP // 16, unpack_b, 0)

    def dot_chunk(c, tw):
        j0 = c * _TCHUNK
        valid = (j0 + _I16()) < cnt
        hrow = _I16()

        def dot_step(i, accs):
            a0, a1 = accs
            d0 = jnp.full((16,), 2 * i, jnp.int32)
            d1 = d0 + 1
            a0 = a0 + rows_t[2 * i, pl.ds(j0, 16)] * plsc.load_gather(tw, [hrow, d0])
            a1 = a1 + rows_t[2 * i + 1, pl.ds(j0, 16)] * plsc.load_gather(tw, [hrow, d1])
            return (a0, a1)

        acc0, acc1 = lax.fori_loop(
            0, _DIM // 2, dot_step,
            (jnp.zeros(16, jnp.float32), jnp.zeros(16, jnp.float32)))
        sig = 1.0 / (1.0 + jnp.exp(-(acc0 + acc1)))
        bv = b_tbl[c, :]
        plsc.store_scatter(buf_a, [bv >> 9, bv & 511], sig, mask=valid)
        return 0

    z16 = jnp.zeros(16, jnp.float32)
    for r in range(32):
        for k in range(32):
            buf_a[r, pl.ds(k * 16, 16)] = z16

    nb = _WLCAP // _TCHUNK
    ha = pltpu.async_copy(t2_hbm.at[b_tbl.at[0]], twin, sem_t)
    for c in range(nb):
        if c % 2 == 0:
            if c + 1 < nb:
                hb = pltpu.async_copy(
                    t2_hbm.at[b_tbl.at[c + 1]], twin_b, sem_u)
            ha.wait()
            dot_chunk(c, twin)
        else:
            if c + 1 < nb:
                ha = pltpu.async_copy(
                    t2_hbm.at[b_tbl.at[c + 1]], twin, sem_t)
            hb.wait()
            dot_chunk(c, twin_b)

    pltpu.sync_copy(buf_a.at[pl.ds(0, 32), :], out_hbm.at[wid])


@jax.jit
def _sc_sweep(xt, idx, t2, edge):
    mesh = plsc.VectorSubcoreMesh(core_axis_name="c", subcore_axis_name="s")
    k = functools.partial(
        pl.kernel,
        mesh=mesh,
        out_type=jax.ShapeDtypeStruct((_NW, 32, 512), jnp.float32),
        scratch_types=[
            pltpu.VMEM((_WLCAP // 128, 128), jnp.int32),
            pltpu.VMEM((_WLCAP // 128, 128), jnp.int32),
            pltpu.VMEM((32,), jnp.int32),
            pltpu.VMEM((16,), jnp.int32),
            pltpu.VMEM((_DIM, _WLCAP), jnp.float32),
            pltpu.VMEM((_TCHUNK, _TEXT_DIM), jnp.float32),
            pltpu.VMEM((_TCHUNK, _TEXT_DIM), jnp.float32),
            pltpu.VMEM((_WLCAP // _TCHUNK, _TCHUNK), jnp.int32),
            pltpu.VMEM((_DIM, _WCOLS * 128), jnp.float32),
            pltpu.VMEM((_DIM, _WCOLS * 128), jnp.float32),
            pltpu.VMEM((_DIM, _EDGEN), jnp.float32),
            pltpu.VMEM((_BCH,), jnp.int32),
            pltpu.SemaphoreType.DMA,
            pltpu.SemaphoreType.DMA,
            pltpu.SemaphoreType.DMA,
            pltpu.SemaphoreType.DMA,
            pltpu.SemaphoreType.DMA,
        ],
        compiler_params=pltpu.CompilerParams(needs_layout_passes=False),
    )(_sc_sweep_kernel)
    return k(xt, idx, t2, edge)


_TCB = 2048


def _tc_t2_kernel(prompt_ref, w_text_ref, w_cls_ref, out_ref):
    t = lax.dot_general(
        prompt_ref[...], w_text_ref[...],
        dimension_numbers=(((1,), (1,)), ((), ())),
        preferred_element_type=jnp.float32,
    )
    out_ref[:, : _DIM] = t * w_cls_ref[...]
    out_ref[:, _DIM:] = jnp.zeros((_TCB, _TEXT_DIM - _DIM), jnp.float32)


@jax.jit
def _tc_t2(prompt, w_text, w_cls):
    grid = _BATCH // _TCB
    return pl.pallas_call(
        _tc_t2_kernel,
        grid=(grid,),
        in_specs=[
            pl.BlockSpec((_TCB, _TEXT_DIM), lambda i: (i, 0)),
            pl.BlockSpec((_DIM, _TEXT_DIM), lambda i: (0, 0)),
            pl.BlockSpec((1, _DIM), lambda i: (0, 0)),
        ],
        out_specs=pl.BlockSpec((_TCB, _TEXT_DIM), lambda i: (i, 0)),
        out_shape=jax.ShapeDtypeStruct((_BATCH, _TEXT_DIM), jnp.float32),
    )(prompt, w_text, w_cls)


def kernel(model_id, prompt_embedding, model_embed_table, W_text, W_cls):
    idx = model_id.astype(jnp.int32)
    xt = model_embed_table.T
    edge = xt[:, _EDGE0:]
    t2 = _tc_t2(prompt_embedding, W_text, W_cls)
    parts = _sc_sweep(xt, idx, t2, edge)
    return parts.reshape(_NW, _BATCH).sum(axis=0)


# speedup vs baseline: 1.0020x; 1.0020x over previous
import functools

import jax
import jax.numpy as jnp
from jax import lax
from jax.experimental import pallas as pl
from jax.experimental.pallas import tpu as pltpu
from jax.experimental.pallas import tpu_sc as plsc

_NUM_MODELS = 1000000
_DIM = 64
_TEXT_DIM = 128
_BATCH = 16384

_INFO = plsc.get_sparse_core_info()
_NC, _NS = _INFO.num_cores, _INFO.num_subcores
_NW = _NC * _NS
_NTC = (_NUM_MODELS + 127) // 128
_TC_PER_W = (_NTC + _NW - 1) // _NW
_WCOLS = 4
_NWIN = (_TC_PER_W + _WCOLS - 1) // _WCOLS
_WLCAP = 640
_WLG = _WLCAP // 16
_MAXC0 = (_NTC - 1 - _WCOLS) * 128
_EDGE0 = (_NTC - 1) * 128
_EDGEN = _NUM_MODELS - _EDGE0
_BCH = 1024
_TCHUNK = 16

_EDGE_MLOC = 255

_I16 = lambda: lax.iota(jnp.int32, 16)


def _sc_sweep_kernel(xt_hbm, idx_hbm, t2_hbm, edge_hbm, out_hbm,
                     wlp, wlp2, win_j, rtab, rows_t, twin, twin_b, b_tbl,
                     buf_a, buf_b, buf_e, ibuf_a,
                     sem_a, sem_b, sem_i, sem_t, sem_u):
    cidx = lax.axis_index("c")
    sid = lax.axis_index("s")
    wid = sid * _NC + cidx
    lo = wid * _TC_PER_W
    hi = jnp.minimum(lo + _TC_PER_W, _NTC)

    def fire(w, buf, sem):
        c0 = pl.multiple_of(jnp.minimum((lo + w * _WCOLS) * 128, _MAXC0), 128)
        for a in range(8):
            pltpu.async_copy(
                xt_hbm.at[pl.ds(a * 8, 8), pl.ds(c0, _WCOLS * 128)],
                buf.at[pl.ds(a * 8, 8), :], sem)

    fire(0, buf_a, sem_a)
    fire(1, buf_b, sem_b)

    def build_chunk(ib, c, cnt):
        def grp(g, cnt):
            iv = ib[pl.ds(g * 16, 16)]
            mv = iv >> 7
            mask = (mv >= lo) & (mv < hi)
            mloc = jnp.where(mv == _NTC - 1, _EDGE_MLOC, mv - lo)
            p = (mloc << 21) | ((c * _BCH + g * 16 + _I16()) << 7) | (iv & 127)
            pos = jnp.minimum(
                cnt + plsc.cumsum(mask.astype(jnp.int32)) - 1, _WLCAP - 1)
            plsc.store_scatter(wlp, [pos >> 7, pos & 127], p, mask=mask)
            return cnt + plsc.all_reduce_population_count(mask)

        return lax.fori_loop(0, _BCH // 16, grp, cnt)

    cnt = jnp.zeros(16, jnp.int32)
    for c in range(_BATCH // _BCH):
        pltpu.sync_copy(idx_hbm.at[pl.ds(c * _BCH, _BCH)], ibuf_a)
        cnt = build_chunk(ibuf_a, c, cnt)

    def rpass(src, dst, bit):
        def czero(g, z):
            p = src[g >> 3, pl.ds((g & 7) * 16, 16)]
            valid = (g * 16 + _I16()) < cnt
            m0 = (((p >> bit) & 1) == 0) & valid
            return z + plsc.all_reduce_population_count(m0)

        z = lax.fori_loop(0, _WLG, czero, jnp.zeros(16, jnp.int32))

        def scat(g, cs):
            c0, c1 = cs
            p = src[g >> 3, pl.ds((g & 7) * 16, 16)]
            valid = (g * 16 + _I16()) < cnt
            b1 = ((p >> bit) & 1) == 1
            m0 = valid & (~b1)
            m1 = valid & b1
            pos0 = jnp.minimum(c0 + plsc.cumsum(m0.astype(jnp.int32)) - 1,
                               _WLCAP - 1)
            pos1 = jnp.minimum(z + c1 + plsc.cumsum(m1.astype(jnp.int32)) - 1,
                               _WLCAP - 1)
            plsc.store_scatter(dst, [pos0 >> 7, pos0 & 127], p, mask=m0)
            plsc.store_scatter(dst, [pos1 >> 7, pos1 & 127], p, mask=m1)
            return (c0 + plsc.all_reduce_population_count(m0),
                    c1 + plsc.all_reduce_population_count(m1))

        lax.fori_loop(0, _WLG, scat,
                      (jnp.zeros(16, jnp.int32), jnp.zeros(16, jnp.int32)))

    rpass(wlp, wlp2, 26)
    rpass(wlp2, wlp, 27)
    rpass(wlp, wlp2, 28)
    wl = wlp2

    def rcount(g, rc):
        p = wl[g >> 3, pl.ds((g & 7) * 16, 16)]
        valid = (g * 16 + _I16()) < cnt
        rv = (p >> 26) & 7
        return tuple(
            rc[r] + plsc.all_reduce_population_count(valid & (rv == r))
            for r in range(8))

    rc = lax.fori_loop(0, _WLG, rcount,
                       tuple(jnp.zeros(16, jnp.int32) for _ in range(8)))
    rstart = [jnp.zeros(16, jnp.int32)]
    for r in range(8):
        rstart.append(rstart[r] + rc[r])
    rvec = jnp.zeros(16, jnp.int32)
    for r in range(9):
        rvec = jnp.where(_I16() == r, rstart[r], rvec)
    rtab[...] = rvec

    def scan_hits(lo_m, hi_m, g_lo, g_hi):
        def body(g, wcnt):
            wm = wl[g >> 3, pl.ds((g & 7) * 16, 16)] >> 21
            valid = (g * 16 + _I16()) < cnt
            mask = (wm >= lo_m) & (wm < hi_m) & valid
            pos = jnp.minimum(
                wcnt + plsc.cumsum(mask.astype(jnp.int32)) - 1, 31)
            plsc.store_scatter(win_j, [pos], g * 16 + _I16(), mask=mask)
            return wcnt + plsc.all_reduce_population_count(mask)

        return lax.fori_loop(g_lo, g_hi, body, jnp.zeros(16, jnp.int32))

    def extract(buf, wcnt, wbase, edge):
        for g in range(2):
            jv = win_j[pl.ds(g * 16, 16)]
            active = (g * 16 + _I16()) < wcnt
            jv = jnp.where(active, jv, 0)
            p = plsc.load_gather(wl, [jv >> 7, jv & 127])
            if edge:
                colloc = p & 127
            else:
                colloc = (((p >> 21) - wbase) * 128) + (p & 127)
            colloc = jnp.where(active, colloc, 0)

            def dstep(i, _):
                d0 = jnp.full((16,), 2 * i, jnp.int32)
                d1 = d0 + 1
                v0 = plsc.load_gather(buf, [d0, colloc])
                v1 = plsc.load_gather(buf, [d1, colloc])
                plsc.store_scatter(rows_t, [d0, jv], v0, mask=active)
                plsc.store_scatter(rows_t, [d1, jv], v1, mask=active)
                return 0

            lax.fori_loop(0, _DIM // 2, dstep, 0)

    def process(w, buf):
        wlo = lo + w * _WCOLS
        c0 = pl.multiple_of(jnp.minimum(wlo * 128, _MAXC0), 128)
        rsp = jnp.full((16,), w >> 3, jnp.int32)
        g_lo = (plsc.load_gather(rtab, [rsp]) >> 4)[0]
        g_hi = ((plsc.load_gather(rtab, [rsp + 1]) + 15) >> 4)[0]
        wcnt = scan_hits(wlo - lo, jnp.minimum(wlo + _WCOLS, _NTC - 1) - lo,
                         g_lo, g_hi)
        extract(buf, wcnt, c0 // 128 - lo, edge=False)

    def body(i, _):
        w0 = i * 2
        pltpu.make_async_copy(
            xt_hbm.at[:, pl.ds(0, _WCOLS * 128)], buf_a, sem_a).wait()
        process(w0, buf_a)

        @pl.when(w0 + 2 < _NWIN)
        def _():
            fire(w0 + 2, buf_a, sem_a)

        pltpu.make_async_copy(
            xt_hbm.at[:, pl.ds(0, _WCOLS * 128)], buf_b, sem_b).wait()
        process(w0 + 1, buf_b)

        @pl.when(w0 + 3 < _NWIN)
        def _():
            fire(w0 + 3, buf_b, sem_b)

        return 0

    lax.fori_loop(0, _NWIN // 2, body, 0)

    pltpu.sync_copy(edge_hbm, buf_e)
    ecnt = scan_hits(_EDGE_MLOC, _EDGE_MLOC + 1,
                     (rstart[7] >> 4)[0], ((rstart[8] + 15) >> 4)[0])
    extract(buf_e, ecnt, 0, edge=True)

    def unpack_b(g, _):
        j0 = g * 16
        valid = (j0 + _I16()) < cnt
        p = wl[g >> 3, pl.ds((g & 7) * 16, 16)]
        bv = jnp.where(valid, (p >> 7) & 16383, 0)
        b_tbl[g, :] = bv
        return 0

    lax.fori_loop(0, _WLCAP // 16, unpack_b, 0)

    def dot_chunk(c, tw):
        j0 = c * _TCHUNK
        valid = (j0 + _I16()) < cnt
        hrow = _I16()

        def dot_step(i, accs):
            a0, a1 = accs
            d0 = jnp.full((16,), 2 * i, jnp.int32)
            d1 = d0 + 1
            a0 = a0 + rows_t[2 * i, pl.ds(j0, 16)] * plsc.load_gather(tw, [hrow, d0])
            a1 = a1 + rows_t[2 * i + 1, pl.ds(j0, 16)] * plsc.load_gather(tw, [hrow, d1])
            return (a0, a1)

        acc0, acc1 = lax.fori_loop(
            0, _DIM // 2, dot_step,
            (jnp.zeros(16, jnp.float32), jnp.zeros(16, jnp.float32)))
        sig = 1.0 / (1.0 + jnp.exp(-(acc0 + acc1)))
        bv = b_tbl[c, :]
        plsc.store_scatter(buf_a, [bv >> 9, bv & 511], sig, mask=valid)
        return 0

    z16 = jnp.zeros(16, jnp.float32)
    for r in range(32):
        for k in range(32):
            buf_a[r, pl.ds(k * 16, 16)] = z16

    nb = _WLCAP // _TCHUNK
    ha = pltpu.async_copy(t2_hbm.at[b_tbl.at[0]], twin, sem_t)
    for c in range(nb):
        if c % 2 == 0:
            if c + 1 < nb:
                hb = pltpu.async_copy(
                    t2_hbm.at[b_tbl.at[c + 1]], twin_b, sem_u)
            ha.wait()
            dot_chunk(c, twin)
        else:
            if c + 1 < nb:
                ha = pltpu.async_copy(
                    t2_hbm.at[b_tbl.at[c + 1]], twin, sem_t)
            hb.wait()
            dot_chunk(c, twin_b)

    pltpu.sync_copy(buf_a.at[pl.ds(0, 32), :], out_hbm.at[wid])


@jax.jit
def _sc_sweep(xt, idx, t2, edge):
    mesh = plsc.VectorSubcoreMesh(core_axis_name="c", subcore_axis_name="s")
    k = functools.partial(
        pl.kernel,
        mesh=mesh,
        out_type=jax.ShapeDtypeStruct((_NW, 32, 512), jnp.float32),
        scratch_types=[
            pltpu.VMEM((_WLCAP // 128, 128), jnp.int32),
            pltpu.VMEM((_WLCAP // 128, 128), jnp.int32),
            pltpu.VMEM((32,), jnp.int32),
            pltpu.VMEM((16,), jnp.int32),
            pltpu.VMEM((_DIM, _WLCAP), jnp.float32),
            pltpu.VMEM((_TCHUNK, _TEXT_DIM), jnp.float32),
            pltpu.VMEM((_TCHUNK, _TEXT_DIM), jnp.float32),
            pltpu.VMEM((_WLCAP // _TCHUNK, _TCHUNK), jnp.int32),
            pltpu.VMEM((_DIM, _WCOLS * 128), jnp.float32),
            pltpu.VMEM((_DIM, _WCOLS * 128), jnp.float32),
            pltpu.VMEM((_DIM, _EDGEN), jnp.float32),
            pltpu.VMEM((_BCH,), jnp.int32),
            pltpu.SemaphoreType.DMA,
            pltpu.SemaphoreType.DMA,
            pltpu.SemaphoreType.DMA,
            pltpu.SemaphoreType.DMA,
            pltpu.SemaphoreType.DMA,
        ],
        compiler_params=pltpu.CompilerParams(needs_layout_passes=False),
    )(_sc_sweep_kernel)
    return k(xt, idx, t2, edge)


_TCB = 2048


def _tc_t2_kernel(prompt_ref, w_text_ref, w_cls_ref, out_ref):
    t = lax.dot_general(
        prompt_ref[...], w_text_ref[...],
        dimension_numbers=(((1,), (1,)), ((), ())),
        preferred_element_type=jnp.float32,
    )
    out_ref[:, : _DIM] = t * w_cls_ref[...]
    out_ref[:, _DIM:] = jnp.zeros((_TCB, _TEXT_DIM - _DIM), jnp.float32)


@jax.jit
def _tc_t2(prompt, w_text, w_cls):
    grid = _BATCH // _TCB
    return pl.pallas_call(
        _tc_t2_kernel,
        grid=(grid,),
        in_specs=[
            pl.BlockSpec((_TCB, _TEXT_DIM), lambda i: (i, 0)),
            pl.BlockSpec((_DIM, _TEXT_DIM), lambda i: (0, 0)),
            pl.BlockSpec((1, _DIM), lambda i: (0, 0)),
        ],
        out_specs=pl.BlockSpec((_TCB, _TEXT_DIM), lambda i: (i, 0)),
        out_shape=jax.ShapeDtypeStruct((_BATCH, _TEXT_DIM), jnp.float32),
    )(prompt, w_text, w_cls)


def kernel(model_id, prompt_embedding, model_embed_table, W_text, W_cls):
    idx = model_id.astype(jnp.int32)
    xt = model_embed_table.T
    edge = xt[:, _EDGE0:]
    t2 = _tc_t2(prompt_embedding, W_text, W_cls)
    parts = _sc_sweep(xt, idx, t2, edge)
    return parts.reshape(_NW, _BATCH).sum(axis=0)
